# baseline (speedup 1.0000x reference)
import jax
import jax.numpy as jnp
from jax import lax
from jax.experimental import pallas as pl
from jax.experimental.pallas import tpu as pltpu
from jax.experimental.pallas import tpu_sc as plsc

N = 10000
E = 320000
F = 128

NC = 2
NS = 16

NP = 10240
NPAD = NP - N
FH = F // NC
NPT = NP // NS
RCH = 64
NRC = NPT // RCH
B = 128
SUBG = 8
NBUF = 4
EPT = 20480
NCHK = EPT // B
NGRP = NCHK // SUBG
EPAD = EPT - E // NS

_mesh = plsc.VectorSubcoreMesh(
    core_axis_name="c", subcore_axis_name="s", num_cores=NC, num_subcores=NS
)
_sc_params = pltpu.CompilerParams(
    needs_layout_passes=False, use_tc_tiling_on_sc=False
)


def _rsqrt16(d):
    i = lax.bitcast_convert_type(d, jnp.int32)
    i = jnp.int32(0x5F3759DF) - lax.shift_right_logical(i, 1)
    y = lax.bitcast_convert_type(i, jnp.float32)
    for _ in range(4):
        y = y * (1.5 - 0.5 * d * y * y)
    return jnp.where(d > 0.0, y, 0.0)


def _deg_dis_body(rc_hbm, x_hbm, dis_hbm, u0_hbm,
                  rcbuf, ones, dbuf, obuf, tbuf, ubuf, dsh, sem):
    c = lax.axis_index("c")
    s = lax.axis_index("s")
    base = pl.multiple_of(s * NPT, RCH)

    @pl.when(c == 0)
    def _prep():
        def f_ones(i, _):
            ones[pl.ds(i * 16, 16)] = jnp.full((16,), 1.0, jnp.float32)
            return 0

        lax.fori_loop(0, B // 16, f_ones, 0)

        def f_zero(i, _):
            dbuf[pl.ds(i * 16, 16)] = jnp.zeros((16,), jnp.float32)
            return 0

        lax.fori_loop(0, NPT // 16, f_zero, 0)
        pltpu.sync_copy(dbuf, dsh.at[pl.ds(s * NPT, NPT)])

    plsc.subcore_barrier()

    @pl.when(c == 0)
    def _scatter():
        def grp(g, _):
            pltpu.sync_copy(
                rc_hbm.at[s, pl.ds(g * (2 * SUBG), 2 * SUBG), :], rcbuf
            )
            for jj in range(SUBG):
                pltpu.sync_copy(ones, dsh.at[rcbuf.at[2 * jj]], add=True)
            return 0

        lax.fori_loop(0, NGRP, grp, 0)

    plsc.subcore_barrier()

    @pl.when(c == 0)
    def _finish():
        pltpu.sync_copy(dsh.at[pl.ds(s * NPT, NPT)], dbuf)

        def grp(i, _):
            obuf[pl.ds(i * 16, 16)] = _rsqrt16(dbuf[pl.ds(i * 16, 16)])
            return 0

        lax.fori_loop(0, NPT // 16, grp, 0)
        pltpu.sync_copy(obuf, dis_hbm.at[s])

        for c2 in range(NC):
            for k in range(NRC):
                r0 = pl.multiple_of(base + k * RCH, RCH)
                rl = k * RCH
                p0 = pl.multiple_of(
                    s * (NPT // 2) + k * (RCH // 2), RCH // 2
                )
                pltpu.sync_copy(x_hbm.at[c2, pl.ds(p0, RCH // 2), :], tbuf)

                def srow(p, _, rl=rl):
                    for half in range(2):
                        sp = plsc.load_gather(
                            obuf, [jnp.broadcast_to(rl + 2 * p + half, (16,))]
                        )
                        for f in range(FH // 16):
                            o = half * FH + f * 16
                            ubuf[2 * p + half, pl.ds(f * 16, 16)] = (
                                tbuf[p, pl.ds(o, 16)] * sp
                            )
                    return 0

                lax.fori_loop(0, RCH // 2, srow, 0)
                pltpu.sync_copy(ubuf, u0_hbm.at[c2, pl.ds(r0, RCH), :])


_deg_dis = pl.kernel(
    _deg_dis_body,
    out_type=(
        jax.ShapeDtypeStruct((NS, NPT), jnp.float32),
        jax.ShapeDtypeStruct((NC, NP, FH), jnp.float32),
    ),
    mesh=_mesh,
    scratch_types=[
        pltpu.VMEM((2 * SUBG, B), jnp.int32),
        pltpu.VMEM((B,), jnp.float32),
        pltpu.VMEM((NPT,), jnp.float32),
        pltpu.VMEM((NPT,), jnp.float32),
        pltpu.VMEM((RCH // 2, F), jnp.float32),
        pltpu.VMEM((RCH, FH), jnp.float32),
        pltpu.VMEM_SHARED((NP,), jnp.float32),
        pltpu.SemaphoreType.DMA,
    ],
    compiler_params=_sc_params,
)


def _make_prop(recur):
    scale = -2.0 if recur else -1.0

    def body(*refs):
        if recur:
            (u_hbm, prev_hbm, rc_hbm, dis_hbm, out_hbm, uo_hbm,
             ash, rcbuf, tbuf, abuf, obuf, ubuf,
             gb0, gb1, gb2, gb3, disb,
             sg0, sg1, sg2, sg3, ss0, ss1, ss2, ss3) = refs
        else:
            (u_hbm, rc_hbm, dis_hbm, out_hbm, uo_hbm,
             ash, rcbuf, tbuf, abuf, obuf, ubuf,
             gb0, gb1, gb2, gb3, disb,
             sg0, sg1, sg2, sg3, ss0, ss1, ss2, ss3) = refs
            prev_hbm = None

        c = lax.axis_index("c")
        s = lax.axis_index("s")
        base = pl.multiple_of(s * NPT, RCH)
        bufs = (gb0, gb1, gb2, gb3)
        gsems = (sg0, sg1, sg2, sg3)
        ssems = (ss0, ss1, ss2, ss3)

        pltpu.sync_copy(dis_hbm.at[s], disb)

        def zrow(r, _):
            for f in range(FH // 16):
                ubuf[r, pl.ds(f * 16, 16)] = jnp.zeros((16,), jnp.float32)
            return 0

        lax.fori_loop(0, RCH, zrow, 0)
        for k in range(NRC):
            pltpu.sync_copy(ubuf, ash.at[pl.ds(base + k * RCH, RCH), :])

        plsc.subcore_barrier()

        usrc = u_hbm.at[c]

        def grp(g, _):
            pltpu.sync_copy(
                rc_hbm.at[s, pl.ds(g * (2 * SUBG), 2 * SUBG), :], rcbuf
            )
            g_desc = [None] * SUBG
            s_desc = [None] * SUBG
            g_desc[0] = pltpu.async_copy(
                usrc.at[rcbuf.at[0]], bufs[0], gsems[0]
            )
            for jj in range(SUBG):
                b = jj % NBUF
                g_desc[jj].wait()
                s_desc[jj] = pltpu.async_copy(
                    bufs[b], ash.at[rcbuf.at[2 * jj + 1]], ssems[b], add=True
                )
                nxt = jj + 1
                if nxt < SUBG:
                    if nxt >= NBUF:
                        s_desc[nxt - NBUF].wait()
                    g_desc[nxt] = pltpu.async_copy(
                        usrc.at[rcbuf.at[2 * nxt]], bufs[nxt % NBUF],
                        gsems[nxt % NBUF],
                    )
            for jj in range(SUBG - NBUF, SUBG):
                s_desc[jj].wait()
            return 0

        lax.fori_loop(0, NGRP, grp, 0)

        plsc.subcore_barrier()

        for k in range(NRC):
            r0 = pl.multiple_of(base + k * RCH, RCH)
            rl = k * RCH
            p0 = pl.multiple_of(s * (NPT // 2) + k * (RCH // 2), RCH // 2)
            pltpu.sync_copy(ash.at[pl.ds(r0, RCH), :], abuf)
            if recur:
                pltpu.sync_copy(prev_hbm.at[c, pl.ds(p0, RCH // 2), :], tbuf)

            def erow(p, _, rl=rl):
                for half in range(2):
                    sp = plsc.load_gather(
                        disb, [jnp.broadcast_to(rl + 2 * p + half, (16,))]
                    )
                    for f in range(FH // 16):
                        o = half * FH + f * 16
                        v = abuf[2 * p + half, pl.ds(f * 16, 16)] * (sp * scale)
                        if recur:
                            v = v - tbuf[p, pl.ds(o, 16)]
                        obuf[p, pl.ds(o, 16)] = v
                        ubuf[2 * p + half, pl.ds(f * 16, 16)] = v * sp
                return 0

            lax.fori_loop(0, RCH // 2, erow, 0)
            pltpu.sync_copy(obuf, out_hbm.at[c, pl.ds(p0, RCH // 2), :])
            pltpu.sync_copy(ubuf, uo_hbm.at[c, pl.ds(r0, RCH), :])

    return pl.kernel(
        body,
        out_type=(
            jax.ShapeDtypeStruct((NC, NP // 2, F), jnp.float32),
            jax.ShapeDtypeStruct((NC, NP, FH), jnp.float32),
        ),
        mesh=_mesh,
        scratch_types=[
            pltpu.VMEM_SHARED((NP, FH), jnp.float32),
            pltpu.VMEM((2 * SUBG, B), jnp.int32),
            pltpu.VMEM((RCH // 2, F), jnp.float32),
            pltpu.VMEM((RCH, FH), jnp.float32),
            pltpu.VMEM((RCH // 2, F), jnp.float32),
            pltpu.VMEM((RCH, FH), jnp.float32),
            pltpu.VMEM((B, FH), jnp.float32),
            pltpu.VMEM((B, FH), jnp.float32),
            pltpu.VMEM((B, FH), jnp.float32),
            pltpu.VMEM((B, FH), jnp.float32),
            pltpu.VMEM((NPT,), jnp.float32),
            pltpu.SemaphoreType.DMA,
            pltpu.SemaphoreType.DMA,
            pltpu.SemaphoreType.DMA,
            pltpu.SemaphoreType.DMA,
            pltpu.SemaphoreType.DMA,
            pltpu.SemaphoreType.DMA,
            pltpu.SemaphoreType.DMA,
            pltpu.SemaphoreType.DMA,
        ],
        compiler_params=_sc_params,
    )


_prop_plain = _make_prop(False)
_prop_recur = _make_prop(True)


def _tc_layer(t0, t1, t2, t3, W, b, dis2d, last):
    H = W.shape[2]
    BR = 512
    G = NP // BR

    def body(*refs):
        if last:
            (t0r, t1r, t2r, t3r, wr, br, outr) = refs
        else:
            (t0r, t1r, t2r, t3r, wr, br, dr, outr, ur) = refs
        acc = br[...] * jnp.float32(1.0)
        for k, tr in enumerate((t0r, t1r, t2r, t3r)):
            acc = acc + jnp.dot(
                tr[...], wr[k], preferred_element_type=jnp.float32
            )
        if last:
            m = jnp.max(acc, axis=1, keepdims=True)
            z = acc - m
            lse = jnp.log(jnp.sum(jnp.exp(z), axis=1, keepdims=True))
            outr[...] = z - lse
        else:
            r = jnp.maximum(acc, 0.0)
            outr[...] = r
            ur[...] = r * dr[...]

    spec = pl.BlockSpec((BR, F), lambda i: (i, 0))
    in_specs = [
        spec, spec, spec, spec,
        pl.BlockSpec((4, F, H), lambda i: (0, 0, 0)),
        pl.BlockSpec((1, H), lambda i: (0, 0)),
    ]
    args = [_unsplit(t0), _unsplit(t1), _unsplit(t2), _unsplit(t3), W, b]
    if last:
        out_shape = jax.ShapeDtypeStruct((NP, H), jnp.float32)
        out_specs = pl.BlockSpec((BR, H), lambda i: (i, 0))
    else:
        in_specs.append(spec)
        args.append(dis2d)
        out_shape = (
            jax.ShapeDtypeStruct((NP, H), jnp.float32),
            jax.ShapeDtypeStruct((NP, F), jnp.float32),
        )
        out_specs = (
            pl.BlockSpec((BR, H), lambda i: (i, 0)),
            pl.BlockSpec((BR, F), lambda i: (i, 0)),
        )
    return pl.pallas_call(
        body,
        out_shape=out_shape,
        grid=(G,),
        in_specs=in_specs,
        out_specs=out_specs,
    )(*args)


def _split(x2d):
    return jnp.stack(
        [x2d[:, :FH].reshape(NP // 2, F), x2d[:, FH:].reshape(NP // 2, F)]
    )


def _unsplit(slabs):
    return jnp.concatenate(
        [slabs[0].reshape(NP, FH), slabs[1].reshape(NP, FH)], axis=1
    )


def _uhalves(u2d):
    return jnp.stack([u2d[:, :FH], u2d[:, FH:]])


def _pad_edges(idx):
    per = idx.reshape(NS, E // NS)
    fill = N + (jnp.arange(EPAD, dtype=jnp.int32) % NPAD)
    fill = jnp.broadcast_to(fill, (NS, EPAD))
    return jnp.concatenate([per, fill], axis=1).reshape(NS, NCHK, B)


def kernel(x, edge_index, W1, b1, W2, b2, W3, b3):
    row2 = _pad_edges(edge_index[0])
    col2 = _pad_edges(edge_index[1])
    rc = jnp.stack([row2, col2], axis=2).reshape(NS, 2 * NCHK, B)

    xp = jnp.pad(x, ((0, NP - N), (0, 0)))
    xs = _split(xp)
    dis, u0 = _deg_dis(rc, xs)
    dis2d = jnp.broadcast_to(dis.reshape(NP)[:, None], (NP, F))

    def cheb(t0s, u, W, b, last):
        t1, u1 = _prop_plain(u, rc, dis)
        t2, u2 = _prop_recur(u1, t0s, rc, dis)
        t3, _ = _prop_recur(u2, t1, rc, dis)
        return _tc_layer(t0s, t1, t2, t3, W, b.reshape(1, -1), dis2d, last)

    h, uh = cheb(xs, u0, W1, b1, False)
    h, uh = cheb(_split(h), _uhalves(uh), W2, b2, False)
    out = cheb(_split(h), _uhalves(uh), W3, b3, True)
    return out[:N]


# speedup vs baseline: 1.1224x; 1.1224x over previous
import jax
import jax.numpy as jnp
from jax import lax
from jax.experimental import pallas as pl
from jax.experimental.pallas import tpu as pltpu
from jax.experimental.pallas import tpu_sc as plsc

N = 10000
E = 320000
F = 128

NC = 2
NS = 16

NP = 10240
NPAD = NP - N
FH = F // NC
NPT = NP // NS
RCH = 64
NRC = NPT // RCH
B = 128
SUBG = 16
EPT = 20480
NGRP = EPT // (B * SUBG)
EPAD = EPT - E // NS

_mesh = plsc.VectorSubcoreMesh(
    core_axis_name="c", subcore_axis_name="s", num_cores=NC, num_subcores=NS
)
_sc_params = pltpu.CompilerParams(needs_layout_passes=False, use_tc_tiling_on_sc=False)


def _rsqrt16(d):
    i = lax.bitcast_convert_type(d, jnp.int32)
    i = jnp.int32(0x5F3759DF) - lax.shift_right_logical(i, 1)
    y = lax.bitcast_convert_type(i, jnp.float32)
    for _ in range(4):
        y = y * (1.5 - 0.5 * d * y * y)
    return jnp.where(d > 0.0, y, 0.0)


def _deg_dis_body(row_hbm, dis_hbm, rbuf, ones, dbuf, obuf, dsh, sem):
    c = lax.axis_index("c")
    s = lax.axis_index("s")

    @pl.when(c == 0)
    def _prep():
        def f_ones(i, _):
            ones[pl.ds(i * 16, 16)] = jnp.full((16,), 1.0, jnp.float32)
            return 0

        lax.fori_loop(0, B // 16, f_ones, 0)

        def f_zero(i, _):
            dbuf[pl.ds(i * 16, 16)] = jnp.zeros((16,), jnp.float32)
            return 0

        lax.fori_loop(0, NPT // 16, f_zero, 0)
        pltpu.sync_copy(dbuf, dsh.at[pl.ds(s * NPT, NPT)])

    plsc.subcore_barrier()

    @pl.when(c == 0)
    def _scatter():
        def grp(g, _):
            pltpu.sync_copy(
                row_hbm.at[s, pl.ds(g * (2 * SUBG), 2 * SUBG), :], rbuf
            )
            for jj in range(SUBG):
                pltpu.sync_copy(ones, dsh.at[rbuf.at[2 * jj]], add=True)
            return 0

        lax.fori_loop(0, NGRP, grp, 0)

    plsc.subcore_barrier()

    @pl.when(c == 0)
    def _rsqrt():
        pltpu.sync_copy(dsh.at[pl.ds(s * NPT, NPT)], dbuf)

        def grp(i, _):
            obuf[pl.ds(i * 16, 16)] = _rsqrt16(dbuf[pl.ds(i * 16, 16)])
            return 0

        lax.fori_loop(0, NPT // 16, grp, 0)
        pltpu.sync_copy(obuf, dis_hbm.at[s])


_deg_dis = pl.kernel(
    _deg_dis_body,
    out_type=jax.ShapeDtypeStruct((NS, NPT), jnp.float32),
    mesh=_mesh,
    scratch_types=[
        pltpu.VMEM((2 * SUBG, B), jnp.int32),
        pltpu.VMEM((B,), jnp.float32),
        pltpu.VMEM((NPT,), jnp.float32),
        pltpu.VMEM((NPT,), jnp.float32),
        pltpu.VMEM_SHARED((NP,), jnp.float32),
        pltpu.SemaphoreType.DMA,
    ],
    compiler_params=_sc_params,
)


def _make_prop(recur):
    scale = -2.0 if recur else -1.0

    def body(*refs):
        if recur:
            (t_hbm, prev_hbm, row_hbm, dis_hbm, out_hbm,
             tsh, ash, rbuf, tbuf, sbuf, abuf, obuf, gbuf, gbuf2,
             disb, sem, sem2, ssem, ssem2) = refs
        else:
            (t_hbm, row_hbm, dis_hbm, out_hbm,
             tsh, ash, rbuf, tbuf, sbuf, abuf, obuf, gbuf, gbuf2,
             disb, sem, sem2, ssem, ssem2) = refs
            prev_hbm = None

        c = lax.axis_index("c")
        s = lax.axis_index("s")
        base = pl.multiple_of(s * NPT, RCH)

        pltpu.sync_copy(dis_hbm.at[s], disb)

        for k in range(NRC):
            r0 = pl.multiple_of(base + k * RCH, RCH)
            rl = k * RCH
            p0 = pl.multiple_of(s * (NPT // 2) + k * (RCH // 2), RCH // 2)
            pltpu.sync_copy(t_hbm.at[c, pl.ds(p0, RCH // 2), :], tbuf)

            def srow(p, _, rl=rl):
                for half in range(2):
                    sp = plsc.load_gather(
                        disb, [jnp.broadcast_to(rl + 2 * p + half, (16,))]
                    )
                    for f in range(FH // 16):
                        o = half * FH + f * 16
                        sbuf[2 * p + half, pl.ds(f * 16, 16)] = (
                            tbuf[p, pl.ds(o, 16)] * sp
                        )
                return 0

            lax.fori_loop(0, RCH // 2, srow, 0)
            pltpu.sync_copy(sbuf, tsh.at[pl.ds(r0, RCH), :])

        def zrow(r, _):
            for f in range(FH // 16):
                sbuf[r, pl.ds(f * 16, 16)] = jnp.zeros((16,), jnp.float32)
            return 0

        lax.fori_loop(0, RCH, zrow, 0)
        for k in range(NRC):
            pltpu.sync_copy(sbuf, ash.at[pl.ds(base + k * RCH, RCH), :])

        plsc.subcore_barrier()

        def grp(g, _):
            pltpu.sync_copy(
                row_hbm.at[s, pl.ds(g * (2 * SUBG), 2 * SUBG), :], rbuf
            )
            bufs = (gbuf, gbuf2)
            gsems = (sem, sem2)
            ssems = (ssem, ssem2)
            g_desc = [None] * SUBG
            s_desc = [None] * SUBG
            g_desc[0] = pltpu.async_copy(
                tsh.at[rbuf.at[0]], bufs[0], gsems[0]
            )
            for jj in range(SUBG):
                b = jj % 2
                g_desc[jj].wait()
                s_desc[jj] = pltpu.async_copy(
                    bufs[b], ash.at[rbuf.at[2 * jj + 1]], ssems[b], add=True
                )
                if jj + 1 < SUBG:
                    if jj >= 1:
                        s_desc[jj - 1].wait()
                    g_desc[jj + 1] = pltpu.async_copy(
                        tsh.at[rbuf.at[2 * (jj + 1)]], bufs[(jj + 1) % 2],
                        gsems[(jj + 1) % 2],
                    )
            s_desc[SUBG - 2].wait()
            s_desc[SUBG - 1].wait()
            return 0

        lax.fori_loop(0, NGRP, grp, 0)

        plsc.subcore_barrier()

        for k in range(NRC):
            r0 = pl.multiple_of(base + k * RCH, RCH)
            rl = k * RCH
            p0 = pl.multiple_of(s * (NPT // 2) + k * (RCH // 2), RCH // 2)
            pltpu.sync_copy(ash.at[pl.ds(r0, RCH), :], abuf)
            if recur:
                pltpu.sync_copy(prev_hbm.at[c, pl.ds(p0, RCH // 2), :], tbuf)

            def erow(p, _, rl=rl):
                for half in range(2):
                    sp = plsc.load_gather(
                        disb, [jnp.broadcast_to(rl + 2 * p + half, (16,))]
                    )
                    sp = sp * scale
                    for f in range(FH // 16):
                        o = half * FH + f * 16
                        v = abuf[2 * p + half, pl.ds(f * 16, 16)] * sp
                        if recur:
                            v = v - tbuf[p, pl.ds(o, 16)]
                        obuf[p, pl.ds(o, 16)] = v
                return 0

            lax.fori_loop(0, RCH // 2, erow, 0)
            pltpu.sync_copy(obuf, out_hbm.at[c, pl.ds(p0, RCH // 2), :])

    return pl.kernel(
        body,
        out_type=jax.ShapeDtypeStruct((NC, NP // 2, F), jnp.float32),
        mesh=_mesh,
        scratch_types=[
            pltpu.VMEM_SHARED((NP, FH), jnp.float32),
            pltpu.VMEM_SHARED((NP, FH), jnp.float32),
            pltpu.VMEM((2 * SUBG, B), jnp.int32),
            pltpu.VMEM((RCH // 2, F), jnp.float32),
            pltpu.VMEM((RCH, FH), jnp.float32),
            pltpu.VMEM((RCH, FH), jnp.float32),
            pltpu.VMEM((RCH // 2, F), jnp.float32),
            pltpu.VMEM((B, FH), jnp.float32),
            pltpu.VMEM((B, FH), jnp.float32),
            pltpu.VMEM((NPT,), jnp.float32),
            pltpu.SemaphoreType.DMA,
            pltpu.SemaphoreType.DMA,
            pltpu.SemaphoreType.DMA,
            pltpu.SemaphoreType.DMA,
        ],
        compiler_params=_sc_params,
    )


_prop_plain = _make_prop(False)
_prop_recur = _make_prop(True)


def _tc_layer(t0, t1, t2, t3, W, b, last):
    H = W.shape[2]
    BR = 512
    G = NP // BR

    def body(t0r, t1r, t2r, t3r, wr, br, outr):
        acc = br[...] * jnp.float32(1.0)
        for k, tr in enumerate((t0r, t1r, t2r, t3r)):
            acc = acc + jnp.dot(
                tr[...], wr[k], preferred_element_type=jnp.float32
            )
        if last:
            m = jnp.max(acc, axis=1, keepdims=True)
            z = acc - m
            lse = jnp.log(jnp.sum(jnp.exp(z), axis=1, keepdims=True))
            outr[...] = z - lse
        else:
            outr[...] = jnp.maximum(acc, 0.0)

    spec = pl.BlockSpec((BR, F), lambda i: (i, 0))
    return pl.pallas_call(
        body,
        out_shape=jax.ShapeDtypeStruct((NP, H), jnp.float32),
        grid=(G,),
        in_specs=[
            spec, spec, spec, spec,
            pl.BlockSpec((4, F, H), lambda i: (0, 0, 0)),
            pl.BlockSpec((1, H), lambda i: (0, 0)),
        ],
        out_specs=pl.BlockSpec((BR, H), lambda i: (i, 0)),
    )(_unsplit(t0), _unsplit(t1), _unsplit(t2), _unsplit(t3), W, b)


def _split(x2d):
    return jnp.stack(
        [x2d[:, :FH].reshape(NP // 2, F), x2d[:, FH:].reshape(NP // 2, F)]
    )


def _unsplit(slabs):
    return jnp.concatenate(
        [slabs[0].reshape(NP, FH), slabs[1].reshape(NP, FH)], axis=1
    )


def _pad_edges(idx):
    per = idx.reshape(NS, E // NS)
    fill = N + (jnp.arange(EPAD, dtype=jnp.int32) % NPAD)
    fill = jnp.broadcast_to(fill, (NS, EPAD))
    return jnp.concatenate([per, fill], axis=1).reshape(NS, EPT // B, B)


def kernel(x, edge_index, W1, b1, W2, b2, W3, b3):
    row2 = _pad_edges(edge_index[0])
    col2 = _pad_edges(edge_index[1])
    rc = jnp.stack([row2, col2], axis=2).reshape(NS, 2 * (EPT // B), B)

    dis = _deg_dis(rc)
    xp = _split(jnp.pad(x, ((0, NP - N), (0, 0))))

    def cheb(t, W, b, last):
        t0 = t
        t1 = _prop_plain(t0, rc, dis)
        t2 = _prop_recur(t1, t0, rc, dis)
        t3 = _prop_recur(t2, t1, rc, dis)
        return _tc_layer(t0, t1, t2, t3, W, b.reshape(1, -1), last)

    h = cheb(xp, W1, b1, False)
    h = cheb(_split(h), W2, b2, False)
    out = cheb(_split(h), W3, b3, True)
    return out[:N]
